# baseline (speedup 1.0000x reference)
import functools

import jax
import jax.numpy as jnp
from jax import lax
from jax.experimental import pallas as pl
from jax.experimental.pallas import tpu as pltpu
from jax.experimental.pallas import tpu_sc as plsc

N = 10000
D = 128
NC = 2
NS = 16
NW = NC * NS
B = 128
N_PAD = 10240
ROWS = N_PAD // NS


def _unpack_chunk(packv, jj, srcb, dstb):
    for k in range(B // 16):
        v = packv[jj, pl.ds(16 * k, 16)]
        srcb[pl.ds(16 * k, 16)] = lax.bitwise_and(v, 0xFFFF)
        dstb[pl.ds(16 * k, 16)] = lax.shift_right_logical(v, 16)


def _make_aggregate(chunks):
    mesh = plsc.VectorSubcoreMesh(core_axis_name="c", subcore_axis_name="s")
    T = chunks // 2

    def body(feats, packi, zf, psum, acc, packv, rows0, rows1,
             srcb0, dstb0, srcb1, dstb1, g0, g1, s0, s1):
        c = lax.axis_index("c")
        s = lax.axis_index("s")
        wid = c * NS + s
        pltpu.sync_copy(zf, acc.at[pl.ds(s * ROWS, ROWS)])
        pltpu.sync_copy(packi.at[wid], packv)
        plsc.subcore_barrier()

        _unpack_chunk(packv, 0, srcb0, dstb0)
        pltpu.async_copy(feats.at[srcb0], rows0, g0)
        _unpack_chunk(packv, 1, srcb1, dstb1)
        pltpu.async_copy(feats.at[srcb1], rows1, g1)
        pltpu.make_async_copy(feats.at[srcb0], rows0, g0).wait()
        pltpu.async_copy(rows0, acc.at[dstb0], s0, add=True)
        pltpu.make_async_copy(rows0, acc.at[dstb0], s0).wait()
        _unpack_chunk(packv, 2, srcb0, dstb0)
        pltpu.async_copy(feats.at[srcb0], rows0, g0)
        pltpu.make_async_copy(feats.at[srcb1], rows1, g1).wait()
        pltpu.async_copy(rows1, acc.at[dstb1], s1, add=True)

        def step(t, carry):
            a = 2 * t
            pltpu.make_async_copy(rows1, acc.at[dstb1], s1).wait()
            _unpack_chunk(packv, a + 1, srcb1, dstb1)
            pltpu.async_copy(feats.at[srcb1], rows1, g1)
            pltpu.make_async_copy(feats.at[srcb0], rows0, g0).wait()
            pltpu.async_copy(rows0, acc.at[dstb0], s0, add=True)
            pltpu.make_async_copy(rows0, acc.at[dstb0], s0).wait()
            _unpack_chunk(packv, jnp.minimum(a + 2, chunks - 1), srcb0, dstb0)

            @pl.when(t < T - 1)
            def _():
                pltpu.async_copy(feats.at[srcb0], rows0, g0)

            pltpu.make_async_copy(feats.at[srcb1], rows1, g1).wait()
            pltpu.async_copy(rows1, acc.at[dstb1], s1, add=True)
            return carry

        lax.fori_loop(1, T, step, 0)
        pltpu.make_async_copy(rows1, acc.at[dstb1], s1).wait()
        plsc.subcore_barrier()
        pltpu.sync_copy(acc.at[pl.ds(s * ROWS, ROWS)],
                        psum.at[c].at[pl.ds(s * ROWS, ROWS)])

    return pl.kernel(
        body,
        out_type=jax.ShapeDtypeStruct((NC, N_PAD, D), jnp.float32),
        mesh=mesh,
        scratch_types=[
            pltpu.VMEM_SHARED((N_PAD, D), jnp.float32),
            pltpu.VMEM((chunks, B), jnp.int32),
            pltpu.VMEM((B, D), jnp.float32),
            pltpu.VMEM((B, D), jnp.float32),
            pltpu.VMEM((B,), jnp.int32),
            pltpu.VMEM((B,), jnp.int32),
            pltpu.VMEM((B,), jnp.int32),
            pltpu.VMEM((B,), jnp.int32),
            pltpu.SemaphoreType.DMA,
            pltpu.SemaphoreType.DMA,
            pltpu.SemaphoreType.DMA,
            pltpu.SemaphoreType.DMA,
        ],
    )


def _make_counts(chunks):
    mesh = plsc.VectorSubcoreMesh(core_axis_name="c", subcore_axis_name="s")

    def body(dsti, zc, ones_in, pcnt, cacc, dstv, onesv):
        c = lax.axis_index("c")
        s = lax.axis_index("s")
        wid = c * NS + s
        pltpu.sync_copy(zc, cacc.at[pl.ds(s * ROWS, ROWS)])
        pltpu.sync_copy(ones_in, onesv)
        pltpu.sync_copy(dsti.at[wid], dstv)
        plsc.subcore_barrier()

        def step(j, carry):
            pltpu.sync_copy(onesv, cacc.at[dstv.at[j]], add=True)
            return carry

        lax.fori_loop(0, chunks, step, 0)
        plsc.subcore_barrier()
        pltpu.sync_copy(cacc.at[pl.ds(s * ROWS, ROWS)],
                        pcnt.at[c].at[pl.ds(s * ROWS, ROWS)])

    return pl.kernel(
        body,
        out_type=jax.ShapeDtypeStruct((NC, N_PAD, D), jnp.float32),
        mesh=mesh,
        scratch_types=[
            pltpu.VMEM_SHARED((N_PAD, D), jnp.float32),
            pltpu.VMEM((chunks, B), jnp.int32),
            pltpu.VMEM((B, D), jnp.float32),
        ],
    )


def _dense_body(act, p0r, p1r, c0r, c1r, xr, wlr, blr, wrr, outr):
    cnt = c0r[...][:, 0:1] + c1r[...][:, 0:1]
    inv = 1.0 / jnp.maximum(cnt, 1.0)
    mean = (p0r[...] + p1r[...]) * inv
    y = (jnp.dot(mean, wlr[...], preferred_element_type=jnp.float32)
         + jnp.dot(xr[...], wrr[...], preferred_element_type=jnp.float32)
         + blr[...])
    if act:
        y = jnp.where(y > 0.0, y, jnp.exp(jnp.minimum(y, 0.0)) - 1.0)
    outr[...] = y


def _dense(p0, p1, c0, c1, x, Wl, bl, Wr, act):
    bn = 1000
    grid = (N // bn,)
    row_spec = pl.BlockSpec((bn, D), lambda i: (i, 0))
    w_spec = pl.BlockSpec((D, D), lambda i: (0, 0))
    b_spec = pl.BlockSpec((1, D), lambda i: (0, 0))
    return pl.pallas_call(
        functools.partial(_dense_body, act),
        grid=grid,
        in_specs=[row_spec, row_spec, row_spec, row_spec, row_spec,
                  w_spec, b_spec, w_spec],
        out_specs=row_spec,
        out_shape=jax.ShapeDtypeStruct((N, D), jnp.float32),
    )(p0, p1, c0, c1, x, Wl, bl.reshape(1, D), Wr)


def kernel(x, edge_index, W1l, b1l, W1r, W2l, b2l, W2r):
    src = edge_index[0]
    dst = edge_index[1]
    e = src.shape[0]
    chunks = -(-e // (NW * B))
    chunks = 2 * (-(-chunks // 2))
    pad = chunks * NW * B - e
    if pad:
        src = jnp.concatenate([src, jnp.zeros((pad,), jnp.int32)])
        dst = jnp.concatenate([dst, jnp.full((pad,), N, jnp.int32)])
    packed = (dst * 65536 + src).reshape(NW, chunks, B)
    dst3 = dst.reshape(NW, chunks, B)
    zf = jnp.zeros((ROWS, D), jnp.float32)
    ones = jnp.ones((B, D), jnp.float32)

    aggregate = _make_aggregate(chunks)
    counts = _make_counts(chunks)

    pcnt = counts(dst3, zf, ones)
    psum = aggregate(x, packed, zf)
    c0 = pcnt[0, :N]
    c1 = pcnt[1, :N]
    h = _dense(psum[0, :N], psum[1, :N], c0, c1, x, W1l, b1l, W1r, act=True)
    psum2 = aggregate(h, packed, zf)
    return _dense(psum2[0, :N], psum2[1, :N], c0, c1, h, W2l, b2l, W2r,
                  act=False)


# speedup vs baseline: 1.8615x; 1.8615x over previous
import functools

import jax
import jax.numpy as jnp
from jax import lax
from jax.experimental import pallas as pl
from jax.experimental.pallas import tpu as pltpu
from jax.experimental.pallas import tpu_sc as plsc

N = 10000
D = 128
NC = 2
NS = 16
NW = NC * NS
B = 128
N_PAD = 10240
ROWS = N_PAD // NS
SPLIT = 0.65


def _make_aggregate(ch0, ch1):
    mesh = plsc.VectorSubcoreMesh(core_axis_name="c", subcore_axis_name="s")
    chm = max(ch0, ch1)

    def body(feats, srci, dsti, zf, psum, acc, srcv, dstv, rows0, sem0):
        c = lax.axis_index("c")
        s = lax.axis_index("s")
        wid = c * NS + s
        nch = jnp.where(c == 0, ch0, ch1)
        pltpu.sync_copy(zf, acc.at[pl.ds(s * ROWS, ROWS)])
        pltpu.sync_copy(srci.at[wid], srcv)
        pltpu.sync_copy(dsti.at[wid], dstv)
        plsc.subcore_barrier()

        def step(j, carry):
            pltpu.async_copy(feats.at[srcv.at[j]], rows0, sem0).wait()
            pltpu.sync_copy(rows0, acc.at[dstv.at[j]], add=True)
            return carry

        lax.fori_loop(0, nch, step, 0)
        plsc.subcore_barrier()
        pltpu.sync_copy(acc.at[pl.ds(s * ROWS, ROWS)],
                        psum.at[c].at[pl.ds(s * ROWS, ROWS)])

    return pl.kernel(
        body,
        out_type=jax.ShapeDtypeStruct((NC, N_PAD, D), jnp.float32),
        mesh=mesh,
        scratch_types=[
            pltpu.VMEM_SHARED((N_PAD, D), jnp.float32),
            pltpu.VMEM((chm, B), jnp.int32),
            pltpu.VMEM((chm, B), jnp.int32),
            pltpu.VMEM((B, D), jnp.float32),
            pltpu.SemaphoreType.DMA,
        ],
    )


def _make_counts(chunks):
    mesh = plsc.VectorSubcoreMesh(core_axis_name="c", subcore_axis_name="s")

    def body(dsti, zc, ones_in, pcnt, cacc, dstv, onesv):
        c = lax.axis_index("c")
        s = lax.axis_index("s")
        wid = c * NS + s
        pltpu.sync_copy(zc, cacc.at[pl.ds(s * ROWS, ROWS)])
        pltpu.sync_copy(ones_in, onesv)
        pltpu.sync_copy(dsti.at[wid], dstv)
        plsc.subcore_barrier()

        def step(j, carry):
            pltpu.sync_copy(onesv, cacc.at[dstv.at[j]], add=True)
            return carry

        lax.fori_loop(0, chunks, step, 0)
        plsc.subcore_barrier()
        pltpu.sync_copy(cacc.at[pl.ds(s * ROWS, ROWS)],
                        pcnt.at[c].at[pl.ds(s * ROWS, ROWS)])

    return pl.kernel(
        body,
        out_type=jax.ShapeDtypeStruct((NC, N_PAD, D), jnp.float32),
        mesh=mesh,
        scratch_types=[
            pltpu.VMEM_SHARED((N_PAD, D), jnp.float32),
            pltpu.VMEM((chunks, B), jnp.int32),
            pltpu.VMEM((B, D), jnp.float32),
        ],
    )


def _dense_body(act, p0r, p1r, c0r, c1r, xr, wlr, blr, wrr, outr):
    cnt = c0r[...][:, 0:1] + c1r[...][:, 0:1]
    inv = 1.0 / jnp.maximum(cnt, 1.0)
    mean = (p0r[...] + p1r[...]) * inv
    y = (jnp.dot(mean, wlr[...], preferred_element_type=jnp.float32)
         + jnp.dot(xr[...], wrr[...], preferred_element_type=jnp.float32)
         + blr[...])
    if act:
        y = jnp.where(y > 0.0, y, jnp.exp(jnp.minimum(y, 0.0)) - 1.0)
    outr[...] = y


def _dense(p0, p1, c0, c1, x, Wl, bl, Wr, act):
    bn = 1000
    grid = (N // bn,)
    row_spec = pl.BlockSpec((bn, D), lambda i: (i, 0))
    w_spec = pl.BlockSpec((D, D), lambda i: (0, 0))
    b_spec = pl.BlockSpec((1, D), lambda i: (0, 0))
    return pl.pallas_call(
        functools.partial(_dense_body, act),
        grid=grid,
        in_specs=[row_spec, row_spec, row_spec, row_spec, row_spec,
                  w_spec, b_spec, w_spec],
        out_specs=row_spec,
        out_shape=jax.ShapeDtypeStruct((N, D), jnp.float32),
    )(p0, p1, c0, c1, x, Wl, bl.reshape(1, D), Wr)


def _pad_edges(src, dst, total):
    pad = total - src.shape[0]
    if pad:
        src = jnp.concatenate([src, jnp.zeros((pad,), jnp.int32)])
        dst = jnp.concatenate([dst, jnp.full((pad,), N, jnp.int32)])
    return src, dst


def _split_layout(arr, ch0, ch1):
    chm = max(ch0, ch1)
    n0 = NS * ch0 * B
    a0 = arr[:n0].reshape(NS, ch0, B)
    a1 = arr[n0:].reshape(NS, ch1, B)
    a0 = jnp.pad(a0, ((0, 0), (0, chm - ch0), (0, 0)))
    a1 = jnp.pad(a1, ((0, 0), (0, chm - ch1), (0, 0)))
    return jnp.concatenate([a0, a1], axis=0)


def kernel(x, edge_index, W1l, b1l, W1r, W2l, b2l, W2r):
    src = edge_index[0]
    dst = edge_index[1]
    e = src.shape[0]

    chunks_u = -(-e // (NW * B))
    src_u, dst_u = _pad_edges(src, dst, chunks_u * NW * B)
    dst3u = dst_u.reshape(NW, chunks_u, B)

    total_chunks = -(-e // (NS * B))
    ch0 = int(round(total_chunks * SPLIT))
    ch0 = min(max(ch0, 1), total_chunks - 1)
    ch1 = total_chunks - ch0
    src_a, dst_a = _pad_edges(src, dst, total_chunks * NS * B)
    src3 = _split_layout(src_a, ch0, ch1)
    dst3 = _split_layout(dst_a, ch0, ch1)

    zf = jnp.zeros((ROWS, D), jnp.float32)
    ones = jnp.ones((B, D), jnp.float32)

    aggregate = _make_aggregate(ch0, ch1)
    counts = _make_counts(chunks_u)

    pcnt = counts(dst3u, zf, ones)
    psum = aggregate(x, src3, dst3, zf)
    c0 = pcnt[0, :N]
    c1 = pcnt[1, :N]
    h = _dense(psum[0, :N], psum[1, :N], c0, c1, x, W1l, b1l, W1r, act=True)
    psum2 = aggregate(h, src3, dst3, zf)
    return _dense(psum2[0, :N], psum2[1, :N], c0, c1, h, W2l, b2l, W2r,
                  act=False)


# speedup vs baseline: 1.9133x; 1.0279x over previous
import functools

import jax
import jax.numpy as jnp
from jax import lax
from jax.experimental import pallas as pl
from jax.experimental.pallas import tpu as pltpu
from jax.experimental.pallas import tpu_sc as plsc

N = 10000
D = 128
NC = 2
NS = 16
NW = NC * NS
B = 128
N_PAD = 10240
ROWS = N_PAD // NS
SPLIT = 0.65


def _make_aggregate(ch0, ch1):
    mesh = plsc.VectorSubcoreMesh(core_axis_name="c", subcore_axis_name="s")
    chm = max(ch0, ch1)

    def body(feats, srci, dsti, zf, psum, acc, srcv, dstv, rows0, sem0):
        c = lax.axis_index("c")
        s = lax.axis_index("s")
        wid = c * NS + s
        nch = jnp.where(c == 0, ch0, ch1)
        pltpu.sync_copy(zf, acc.at[pl.ds(s * ROWS, ROWS)])
        pltpu.sync_copy(srci.at[wid], srcv)
        pltpu.sync_copy(dsti.at[wid], dstv)
        plsc.subcore_barrier()

        def step(j, carry):
            pltpu.async_copy(feats.at[srcv.at[j]], rows0, sem0).wait()
            pltpu.sync_copy(rows0, acc.at[dstv.at[j]], add=True)
            return carry

        lax.fori_loop(0, nch, step, 0)
        plsc.subcore_barrier()
        pltpu.sync_copy(acc.at[pl.ds(s * ROWS, ROWS)],
                        psum.at[c].at[pl.ds(s * ROWS, ROWS)])

    return pl.kernel(
        body,
        out_type=jax.ShapeDtypeStruct((NC, N_PAD, D), jnp.float32),
        mesh=mesh,
        scratch_types=[
            pltpu.VMEM_SHARED((N_PAD, D), jnp.float32),
            pltpu.VMEM((chm, B), jnp.int32),
            pltpu.VMEM((chm, B), jnp.int32),
            pltpu.VMEM((B, D), jnp.float32),
            pltpu.SemaphoreType.DMA,
        ],
    )


def _make_counts(chunks):
    mesh = plsc.VectorSubcoreMesh(core_axis_name="c", subcore_axis_name="s")

    def body(dsti, zc, ones_in, pcnt, cacc, dstv, onesv):
        c = lax.axis_index("c")
        s = lax.axis_index("s")
        wid = c * NS + s
        pltpu.sync_copy(zc, cacc.at[pl.ds(s * ROWS, ROWS)])
        pltpu.sync_copy(ones_in, onesv)
        pltpu.sync_copy(dsti.at[wid], dstv)
        plsc.subcore_barrier()

        def step(j, carry):
            pltpu.sync_copy(onesv, cacc.at[dstv.at[j]], add=True)
            return carry

        lax.fori_loop(0, chunks, step, 0)
        plsc.subcore_barrier()
        pltpu.sync_copy(cacc.at[pl.ds(s * ROWS, ROWS)],
                        pcnt.at[c].at[pl.ds(s * ROWS, ROWS)])

    return pl.kernel(
        body,
        out_type=jax.ShapeDtypeStruct((NC, N_PAD, D), jnp.float32),
        mesh=mesh,
        scratch_types=[
            pltpu.VMEM_SHARED((N_PAD, D), jnp.float32),
            pltpu.VMEM((chunks, B), jnp.int32),
            pltpu.VMEM((B, D), jnp.float32),
        ],
    )


def _dense_body(act, p0r, p1r, c0r, c1r, xr, wlr, blr, wrr, outr):
    cnt = c0r[0][:, 0:1] + c1r[0][:, 0:1]
    inv = 1.0 / jnp.maximum(cnt, 1.0)
    mean = (p0r[0] + p1r[0]) * inv
    y = (jnp.dot(mean, wlr[...], preferred_element_type=jnp.float32)
         + jnp.dot(xr[...], wrr[...], preferred_element_type=jnp.float32)
         + blr[...])
    if act:
        y = jnp.where(y > 0.0, y, jnp.exp(jnp.minimum(y, 0.0)) - 1.0)
    outr[...] = y


def _dense(psum, pcnt, x, Wl, bl, Wr, act):
    bn = 1000
    grid = (N // bn,)
    part0_spec = pl.BlockSpec((1, bn, D), lambda i: (0, i, 0))
    part1_spec = pl.BlockSpec((1, bn, D), lambda i: (1, i, 0))
    row_spec = pl.BlockSpec((bn, D), lambda i: (i, 0))
    w_spec = pl.BlockSpec((D, D), lambda i: (0, 0))
    b_spec = pl.BlockSpec((1, D), lambda i: (0, 0))
    return pl.pallas_call(
        functools.partial(_dense_body, act),
        grid=grid,
        in_specs=[part0_spec, part1_spec, part0_spec, part1_spec, row_spec,
                  w_spec, b_spec, w_spec],
        out_specs=row_spec,
        out_shape=jax.ShapeDtypeStruct((N, D), jnp.float32),
    )(psum, psum, pcnt, pcnt, x, Wl, bl.reshape(1, D), Wr)


def _pad_edges(src, dst, total):
    pad = total - src.shape[0]
    if pad:
        src = jnp.concatenate([src, jnp.zeros((pad,), jnp.int32)])
        dst = jnp.concatenate([dst, jnp.full((pad,), N, jnp.int32)])
    return src, dst


def _split_layout(arr, ch0, ch1):
    chm = max(ch0, ch1)
    n0 = NS * ch0 * B
    a0 = arr[:n0].reshape(NS, ch0, B)
    a1 = arr[n0:].reshape(NS, ch1, B)
    a0 = jnp.pad(a0, ((0, 0), (0, chm - ch0), (0, 0)))
    a1 = jnp.pad(a1, ((0, 0), (0, chm - ch1), (0, 0)))
    return jnp.concatenate([a0, a1], axis=0)


def kernel(x, edge_index, W1l, b1l, W1r, W2l, b2l, W2r):
    src = edge_index[0]
    dst = edge_index[1]
    e = src.shape[0]

    chunks_u = -(-e // (NW * B))
    src_u, dst_u = _pad_edges(src, dst, chunks_u * NW * B)
    dst3u = dst_u.reshape(NW, chunks_u, B)

    total_chunks = -(-e // (NS * B))
    ch0 = int(round(total_chunks * SPLIT))
    ch0 = min(max(ch0, 1), total_chunks - 1)
    ch1 = total_chunks - ch0
    src_a, dst_a = _pad_edges(src, dst, total_chunks * NS * B)
    src3 = _split_layout(src_a, ch0, ch1)
    dst3 = _split_layout(dst_a, ch0, ch1)

    zf = jnp.zeros((ROWS, D), jnp.float32)
    ones = jnp.ones((B, D), jnp.float32)

    aggregate = _make_aggregate(ch0, ch1)
    counts = _make_counts(chunks_u)

    pcnt = counts(dst3u, zf, ones)
    psum = aggregate(x, src3, dst3, zf)
    h = _dense(psum, pcnt, x, W1l, b1l, W1r, act=True)
    psum2 = aggregate(h, src3, dst3, zf)
    return _dense(psum2, pcnt, h, W2l, b2l, W2r, act=False)


# speedup vs baseline: 2.2273x; 1.1641x over previous
import functools

import jax
import jax.numpy as jnp
from jax import lax
from jax.experimental import pallas as pl
from jax.experimental.pallas import tpu as pltpu
from jax.experimental.pallas import tpu_sc as plsc

N = 10000
D = 128
NC = 2
NS = 16
NW = NC * NS
B = 128
N_PAD = 10240
ROWS = N_PAD // NS
SPLIT = 0.71


def _unpack_chunk(packv, jj, srcb, dstb):
    for k in range(B // 16):
        v = packv[jj, pl.ds(16 * k, 16)]
        srcb[pl.ds(16 * k, 16)] = lax.bitwise_and(v, 0xFFFF)
        dstb[pl.ds(16 * k, 16)] = lax.shift_right_logical(v, 16)


def _make_aggregate(ch0, ch1):
    mesh = plsc.VectorSubcoreMesh(core_axis_name="c", subcore_axis_name="s")
    chm = max(ch0, ch1)
    T0 = ch0 // 2

    def body(feats, packi, zf, psum, acc, packv, rows0, rows1,
             srcb0, dstb0, srcb1, dstb1, g0, g1, s0, s1):
        c = lax.axis_index("c")
        s = lax.axis_index("s")
        wid = c * NS + s
        pltpu.sync_copy(zf, acc.at[pl.ds(s * ROWS, ROWS)])
        pltpu.sync_copy(packi.at[wid], packv)
        plsc.subcore_barrier()

        @pl.when(c == 0)
        def _pipelined():
            _unpack_chunk(packv, 0, srcb0, dstb0)
            pltpu.async_copy(feats.at[srcb0], rows0, g0)
            _unpack_chunk(packv, 1, srcb1, dstb1)
            pltpu.async_copy(feats.at[srcb1], rows1, g1)
            pltpu.make_async_copy(feats.at[srcb0], rows0, g0).wait()
            pltpu.async_copy(rows0, acc.at[dstb0], s0, add=True)
            pltpu.make_async_copy(rows0, acc.at[dstb0], s0).wait()
            _unpack_chunk(packv, 2, srcb0, dstb0)
            pltpu.async_copy(feats.at[srcb0], rows0, g0)
            pltpu.make_async_copy(feats.at[srcb1], rows1, g1).wait()
            pltpu.async_copy(rows1, acc.at[dstb1], s1, add=True)

            def step(t, carry):
                a = 2 * t
                pltpu.make_async_copy(rows1, acc.at[dstb1], s1).wait()
                _unpack_chunk(packv, a + 1, srcb1, dstb1)
                pltpu.async_copy(feats.at[srcb1], rows1, g1)
                pltpu.make_async_copy(feats.at[srcb0], rows0, g0).wait()
                pltpu.async_copy(rows0, acc.at[dstb0], s0, add=True)
                pltpu.make_async_copy(rows0, acc.at[dstb0], s0).wait()
                _unpack_chunk(packv, jnp.minimum(a + 2, ch0 - 1),
                              srcb0, dstb0)

                @pl.when(t < T0 - 1)
                def _():
                    pltpu.async_copy(feats.at[srcb0], rows0, g0)

                pltpu.make_async_copy(feats.at[srcb1], rows1, g1).wait()
                pltpu.async_copy(rows1, acc.at[dstb1], s1, add=True)
                return carry

            lax.fori_loop(1, T0, step, 0)
            pltpu.make_async_copy(rows1, acc.at[dstb1], s1).wait()

        @pl.when(c == 1)
        def _simple():
            def step(j, carry):
                _unpack_chunk(packv, j, srcb0, dstb0)
                pltpu.async_copy(feats.at[srcb0], rows0, g0).wait()
                pltpu.sync_copy(rows0, acc.at[dstb0], add=True)
                return carry

            lax.fori_loop(0, ch1, step, 0)

        plsc.subcore_barrier()
        pltpu.sync_copy(acc.at[pl.ds(s * ROWS, ROWS)],
                        psum.at[c].at[pl.ds(s * ROWS, ROWS)])

    return pl.kernel(
        body,
        out_type=jax.ShapeDtypeStruct((NC, N_PAD, D), jnp.float32),
        mesh=mesh,
        scratch_types=[
            pltpu.VMEM_SHARED((N_PAD, D), jnp.float32),
            pltpu.VMEM((chm, B), jnp.int32),
            pltpu.VMEM((B, D), jnp.float32),
            pltpu.VMEM((B, D), jnp.float32),
            pltpu.VMEM((B,), jnp.int32),
            pltpu.VMEM((B,), jnp.int32),
            pltpu.VMEM((B,), jnp.int32),
            pltpu.VMEM((B,), jnp.int32),
            pltpu.SemaphoreType.DMA,
            pltpu.SemaphoreType.DMA,
            pltpu.SemaphoreType.DMA,
            pltpu.SemaphoreType.DMA,
        ],
    )


def _make_counts(chunks):
    mesh = plsc.VectorSubcoreMesh(core_axis_name="c", subcore_axis_name="s")

    def body(dsti, zc, ones_in, pcnt, cacc, dstv, onesv):
        c = lax.axis_index("c")
        s = lax.axis_index("s")
        wid = c * NS + s
        pltpu.sync_copy(zc, cacc.at[pl.ds(s * ROWS, ROWS)])
        pltpu.sync_copy(ones_in, onesv)
        pltpu.sync_copy(dsti.at[wid], dstv)
        plsc.subcore_barrier()

        def step(j, carry):
            pltpu.sync_copy(onesv, cacc.at[dstv.at[j]], add=True)
            return carry

        lax.fori_loop(0, chunks, step, 0)
        plsc.subcore_barrier()
        pltpu.sync_copy(cacc.at[pl.ds(s * ROWS, ROWS)],
                        pcnt.at[c].at[pl.ds(s * ROWS, ROWS)])

    return pl.kernel(
        body,
        out_type=jax.ShapeDtypeStruct((NC, N_PAD, D), jnp.float32),
        mesh=mesh,
        scratch_types=[
            pltpu.VMEM_SHARED((N_PAD, D), jnp.float32),
            pltpu.VMEM((chunks, B), jnp.int32),
            pltpu.VMEM((B, D), jnp.float32),
        ],
    )


def _dense_body(act, p0r, p1r, c0r, c1r, xr, wlr, blr, wrr, outr):
    cnt = c0r[0][:, 0:1] + c1r[0][:, 0:1]
    inv = 1.0 / jnp.maximum(cnt, 1.0)
    mean = (p0r[0] + p1r[0]) * inv
    y = (jnp.dot(mean, wlr[...], preferred_element_type=jnp.float32)
         + jnp.dot(xr[...], wrr[...], preferred_element_type=jnp.float32)
         + blr[...])
    if act:
        y = jnp.where(y > 0.0, y, jnp.exp(jnp.minimum(y, 0.0)) - 1.0)
    outr[...] = y


def _dense(psum, pcnt, x, Wl, bl, Wr, act):
    bn = 1000
    grid = (N // bn,)
    part0_spec = pl.BlockSpec((1, bn, D), lambda i: (0, i, 0))
    part1_spec = pl.BlockSpec((1, bn, D), lambda i: (1, i, 0))
    row_spec = pl.BlockSpec((bn, D), lambda i: (i, 0))
    w_spec = pl.BlockSpec((D, D), lambda i: (0, 0))
    b_spec = pl.BlockSpec((1, D), lambda i: (0, 0))
    return pl.pallas_call(
        functools.partial(_dense_body, act),
        grid=grid,
        in_specs=[part0_spec, part1_spec, part0_spec, part1_spec, row_spec,
                  w_spec, b_spec, w_spec],
        out_specs=row_spec,
        out_shape=jax.ShapeDtypeStruct((N, D), jnp.float32),
    )(psum, psum, pcnt, pcnt, x, Wl, bl.reshape(1, D), Wr)


def _pad_edges(src, dst, total):
    pad = total - src.shape[0]
    if pad:
        src = jnp.concatenate([src, jnp.zeros((pad,), jnp.int32)])
        dst = jnp.concatenate([dst, jnp.full((pad,), N, jnp.int32)])
    return src, dst


def _split_layout(arr, ch0, ch1):
    chm = max(ch0, ch1)
    n0 = NS * ch0 * B
    a0 = arr[:n0].reshape(NS, ch0, B)
    a1 = arr[n0:].reshape(NS, ch1, B)
    a0 = jnp.pad(a0, ((0, 0), (0, chm - ch0), (0, 0)))
    a1 = jnp.pad(a1, ((0, 0), (0, chm - ch1), (0, 0)))
    return jnp.concatenate([a0, a1], axis=0)


def kernel(x, edge_index, W1l, b1l, W1r, W2l, b2l, W2r):
    src = edge_index[0]
    dst = edge_index[1]
    e = src.shape[0]

    chunks_u = -(-e // (NW * B))
    src_u, dst_u = _pad_edges(src, dst, chunks_u * NW * B)
    dst3u = dst_u.reshape(NW, chunks_u, B)

    total_chunks = -(-e // (NS * B))
    ch0 = int(round(total_chunks * SPLIT))
    ch0 = min(max(2 * (ch0 // 2), 4), total_chunks - 1)
    ch1 = total_chunks - ch0
    src_a, dst_a = _pad_edges(src, dst, total_chunks * NS * B)
    packed = _split_layout(dst_a * 65536 + src_a, ch0, ch1)

    zf = jnp.zeros((ROWS, D), jnp.float32)
    ones = jnp.ones((B, D), jnp.float32)

    aggregate = _make_aggregate(ch0, ch1)
    counts = _make_counts(chunks_u)

    pcnt = counts(dst3u, zf, ones)
    psum = aggregate(x, packed, zf)
    h = _dense(psum, pcnt, x, W1l, b1l, W1r, act=True)
    psum2 = aggregate(h, packed, zf)
    return _dense(psum2, pcnt, h, W2l, b2l, W2r, act=False)


# speedup vs baseline: 2.3683x; 1.0633x over previous
import functools

import jax
import jax.numpy as jnp
from jax import lax
from jax.experimental import pallas as pl
from jax.experimental.pallas import tpu as pltpu
from jax.experimental.pallas import tpu_sc as plsc

N = 10000
D = 128
NC = 2
NS = 16
NW = NC * NS
B = 128
N_PAD = 10240
ROWS = N_PAD // NS
SPLIT = 0.76


def _unpack_chunk(packv, jj, srcb, dstb):
    for k in range(B // 16):
        v = packv[jj, pl.ds(16 * k, 16)]
        srcb[pl.ds(16 * k, 16)] = lax.bitwise_and(v, 0xFFFF)
        dstb[pl.ds(16 * k, 16)] = lax.shift_right_logical(v, 16)


def _make_aggregate(ch0, ch1):
    mesh = plsc.VectorSubcoreMesh(core_axis_name="c", subcore_axis_name="s")
    chm = max(ch0, ch1)
    T0 = ch0 // 2

    def body(feats, packi, zf, psum, acc, packv, rows0, rows1,
             srcb0, dstb0, srcb1, dstb1, g0, g1, s0, s1):
        c = lax.axis_index("c")
        s = lax.axis_index("s")
        wid = c * NS + s
        pltpu.sync_copy(zf, acc.at[pl.ds(s * ROWS, ROWS)])
        pltpu.sync_copy(packi.at[wid], packv)
        plsc.subcore_barrier()

        @pl.when(c == 0)
        def _pipelined():
            _unpack_chunk(packv, 0, srcb0, dstb0)
            pltpu.async_copy(feats.at[srcb0], rows0, g0)
            _unpack_chunk(packv, 1, srcb1, dstb1)
            pltpu.async_copy(feats.at[srcb1], rows1, g1)
            pltpu.make_async_copy(feats.at[srcb0], rows0, g0).wait()
            pltpu.async_copy(rows0, acc.at[dstb0], s0, add=True)
            pltpu.make_async_copy(rows0, acc.at[dstb0], s0).wait()
            _unpack_chunk(packv, 2, srcb0, dstb0)
            pltpu.async_copy(feats.at[srcb0], rows0, g0)
            pltpu.make_async_copy(feats.at[srcb1], rows1, g1).wait()
            pltpu.async_copy(rows1, acc.at[dstb1], s1, add=True)

            def step(t, carry):
                a = 2 * t
                pltpu.make_async_copy(rows1, acc.at[dstb1], s1).wait()
                _unpack_chunk(packv, a + 1, srcb1, dstb1)
                pltpu.async_copy(feats.at[srcb1], rows1, g1)
                pltpu.make_async_copy(feats.at[srcb0], rows0, g0).wait()
                pltpu.async_copy(rows0, acc.at[dstb0], s0, add=True)
                pltpu.make_async_copy(rows0, acc.at[dstb0], s0).wait()
                _unpack_chunk(packv, jnp.minimum(a + 2, ch0 - 1),
                              srcb0, dstb0)

                @pl.when(t < T0 - 1)
                def _():
                    pltpu.async_copy(feats.at[srcb0], rows0, g0)

                pltpu.make_async_copy(feats.at[srcb1], rows1, g1).wait()
                pltpu.async_copy(rows1, acc.at[dstb1], s1, add=True)
                return carry

            lax.fori_loop(1, T0, step, 0)
            pltpu.make_async_copy(rows1, acc.at[dstb1], s1).wait()

        @pl.when(c == 1)
        def _simple():
            def step(j, carry):
                _unpack_chunk(packv, j, srcb0, dstb0)
                pltpu.async_copy(feats.at[srcb0], rows0, g0).wait()
                pltpu.sync_copy(rows0, acc.at[dstb0], add=True)
                return carry

            lax.fori_loop(0, ch1, step, 0)

        plsc.subcore_barrier()
        pltpu.sync_copy(acc.at[pl.ds(s * ROWS, ROWS)],
                        psum.at[c].at[pl.ds(s * ROWS, ROWS)])

    return pl.kernel(
        body,
        out_type=jax.ShapeDtypeStruct((NC, N_PAD, D), jnp.float32),
        mesh=mesh,
        scratch_types=[
            pltpu.VMEM_SHARED((N_PAD, D), jnp.float32),
            pltpu.VMEM((chm, B), jnp.int32),
            pltpu.VMEM((B, D), jnp.float32),
            pltpu.VMEM((B, D), jnp.float32),
            pltpu.VMEM((B,), jnp.int32),
            pltpu.VMEM((B,), jnp.int32),
            pltpu.VMEM((B,), jnp.int32),
            pltpu.VMEM((B,), jnp.int32),
            pltpu.SemaphoreType.DMA,
            pltpu.SemaphoreType.DMA,
            pltpu.SemaphoreType.DMA,
            pltpu.SemaphoreType.DMA,
        ],
    )


def _make_counts(chunks):
    mesh = plsc.VectorSubcoreMesh(core_axis_name="c", subcore_axis_name="s")

    def body(dsti, zc, ones_in, pcnt, cacc, dstv, onesv):
        c = lax.axis_index("c")
        s = lax.axis_index("s")
        wid = c * NS + s
        pltpu.sync_copy(zc, cacc.at[pl.ds(s * ROWS, ROWS)])
        pltpu.sync_copy(ones_in, onesv)
        pltpu.sync_copy(dsti.at[wid], dstv)
        plsc.subcore_barrier()

        def step(j, carry):
            pltpu.sync_copy(onesv, cacc.at[dstv.at[j]], add=True)
            return carry

        lax.fori_loop(0, chunks, step, 0)
        plsc.subcore_barrier()
        pltpu.sync_copy(cacc.at[pl.ds(s * ROWS, ROWS)],
                        pcnt.at[c].at[pl.ds(s * ROWS, ROWS)])

    return pl.kernel(
        body,
        out_type=jax.ShapeDtypeStruct((NC, N_PAD, D), jnp.float32),
        mesh=mesh,
        scratch_types=[
            pltpu.VMEM_SHARED((N_PAD, D), jnp.float32),
            pltpu.VMEM((chunks, B), jnp.int32),
            pltpu.VMEM((B, D), jnp.float32),
        ],
    )


def _dense_body(act, p0r, p1r, c0r, c1r, xr, wlr, blr, wrr, outr):
    cnt = c0r[0][:, 0:1] + c1r[0][:, 0:1]
    inv = 1.0 / jnp.maximum(cnt, 1.0)
    mean = (p0r[0] + p1r[0]) * inv
    y = (jnp.dot(mean, wlr[...], preferred_element_type=jnp.float32)
         + jnp.dot(xr[...], wrr[...], preferred_element_type=jnp.float32)
         + blr[...])
    if act:
        y = jnp.where(y > 0.0, y, jnp.exp(jnp.minimum(y, 0.0)) - 1.0)
    outr[...] = y


def _dense(psum, pcnt, x, Wl, bl, Wr, act):
    bn = 1000
    grid = (N // bn,)
    part0_spec = pl.BlockSpec((1, bn, D), lambda i: (0, i, 0))
    part1_spec = pl.BlockSpec((1, bn, D), lambda i: (1, i, 0))
    row_spec = pl.BlockSpec((bn, D), lambda i: (i, 0))
    w_spec = pl.BlockSpec((D, D), lambda i: (0, 0))
    b_spec = pl.BlockSpec((1, D), lambda i: (0, 0))
    return pl.pallas_call(
        functools.partial(_dense_body, act),
        grid=grid,
        in_specs=[part0_spec, part1_spec, part0_spec, part1_spec, row_spec,
                  w_spec, b_spec, w_spec],
        out_specs=row_spec,
        out_shape=jax.ShapeDtypeStruct((N, D), jnp.float32),
    )(psum, psum, pcnt, pcnt, x, Wl, bl.reshape(1, D), Wr)


def _pad_edges(src, dst, total):
    pad = total - src.shape[0]
    if pad:
        src = jnp.concatenate([src, jnp.zeros((pad,), jnp.int32)])
        dst = jnp.concatenate([dst, jnp.full((pad,), N, jnp.int32)])
    return src, dst


def _split_layout(arr, ch0, ch1):
    chm = max(ch0, ch1)
    n0 = NS * ch0 * B
    a0 = arr[:n0].reshape(NS, ch0, B)
    a1 = arr[n0:].reshape(NS, ch1, B)
    a0 = jnp.pad(a0, ((0, 0), (0, chm - ch0), (0, 0)))
    a1 = jnp.pad(a1, ((0, 0), (0, chm - ch1), (0, 0)))
    return jnp.concatenate([a0, a1], axis=0)


def kernel(x, edge_index, W1l, b1l, W1r, W2l, b2l, W2r):
    src = edge_index[0]
    dst = edge_index[1]
    e = src.shape[0]

    chunks_u = -(-e // (NW * B))
    src_u, dst_u = _pad_edges(src, dst, chunks_u * NW * B)
    dst3u = dst_u.reshape(NW, chunks_u, B)

    total_chunks = -(-e // (NS * B))
    ch0 = int(round(total_chunks * SPLIT))
    ch0 = min(max(2 * (ch0 // 2), 4), total_chunks - 1)
    ch1 = total_chunks - ch0
    src_a, dst_a = _pad_edges(src, dst, total_chunks * NS * B)
    packed = _split_layout(dst_a * 65536 + src_a, ch0, ch1)

    zf = jnp.zeros((ROWS, D), jnp.float32)
    ones = jnp.ones((B, D), jnp.float32)

    aggregate = _make_aggregate(ch0, ch1)
    counts = _make_counts(chunks_u)

    pcnt = counts(dst3u, zf, ones)
    psum = aggregate(x, packed, zf)
    h = _dense(psum, pcnt, x, W1l, b1l, W1r, act=True)
    psum2 = aggregate(h, packed, zf)
    return _dense(psum2, pcnt, h, W2l, b2l, W2r, act=False)


# speedup vs baseline: 2.5625x; 1.0820x over previous
import functools

import jax
import jax.numpy as jnp
from jax import lax
from jax.experimental import pallas as pl
from jax.experimental.pallas import tpu as pltpu
from jax.experimental.pallas import tpu_sc as plsc

N = 10000
D = 128
NC = 2
NS = 16
NW = NC * NS
B = 128
N_PAD = 10240
ROWS = N_PAD // NS
SPLIT = 0.76


def _unpack_chunk(packv, jj, srcb, dstb):
    for k in range(B // 16):
        v = packv[jj, pl.ds(16 * k, 16)]
        srcb[pl.ds(16 * k, 16)] = lax.bitwise_and(v, 0xFFFF)
        dstb[pl.ds(16 * k, 16)] = lax.shift_right_logical(v, 16)


def _make_aggregate(ch0, ch1):
    mesh = plsc.VectorSubcoreMesh(core_axis_name="c", subcore_axis_name="s")
    chm = max(ch0, ch1)
    T0 = ch0 // 2

    def body(feats, packi, zf, psum, acc, packv, rows0, rows1,
             srcb0, dstb0, srcb1, dstb1, g0, g1, s0, s1):
        c = lax.axis_index("c")
        s = lax.axis_index("s")
        wid = c * NS + s
        pltpu.sync_copy(zf, acc.at[pl.ds(s * ROWS, ROWS)])
        pltpu.sync_copy(packi.at[wid], packv)
        plsc.subcore_barrier()

        @pl.when(c == 0)
        def _pipelined():
            _unpack_chunk(packv, 0, srcb0, dstb0)
            pltpu.async_copy(feats.at[srcb0], rows0, g0)
            _unpack_chunk(packv, 1, srcb1, dstb1)
            pltpu.async_copy(feats.at[srcb1], rows1, g1)
            pltpu.make_async_copy(feats.at[srcb0], rows0, g0).wait()
            pltpu.async_copy(rows0, acc.at[dstb0], s0, add=True)
            pltpu.make_async_copy(rows0, acc.at[dstb0], s0).wait()
            _unpack_chunk(packv, 2, srcb0, dstb0)
            pltpu.async_copy(feats.at[srcb0], rows0, g0)
            pltpu.make_async_copy(feats.at[srcb1], rows1, g1).wait()
            pltpu.async_copy(rows1, acc.at[dstb1], s1, add=True)

            def step(t, carry):
                a = 2 * t
                pltpu.make_async_copy(rows1, acc.at[dstb1], s1).wait()
                _unpack_chunk(packv, a + 1, srcb1, dstb1)
                pltpu.async_copy(feats.at[srcb1], rows1, g1)
                pltpu.make_async_copy(feats.at[srcb0], rows0, g0).wait()
                pltpu.async_copy(rows0, acc.at[dstb0], s0, add=True)
                pltpu.make_async_copy(rows0, acc.at[dstb0], s0).wait()
                _unpack_chunk(packv, jnp.minimum(a + 2, ch0 - 1),
                              srcb0, dstb0)

                @pl.when(t < T0 - 1)
                def _():
                    pltpu.async_copy(feats.at[srcb0], rows0, g0)

                pltpu.make_async_copy(feats.at[srcb1], rows1, g1).wait()
                pltpu.async_copy(rows1, acc.at[dstb1], s1, add=True)
                return carry

            lax.fori_loop(1, T0, step, 0)
            pltpu.make_async_copy(rows1, acc.at[dstb1], s1).wait()

        @pl.when(c == 1)
        def _simple():
            def step(j, carry):
                _unpack_chunk(packv, j, srcb0, dstb0)
                pltpu.async_copy(feats.at[srcb0], rows0, g0).wait()
                pltpu.sync_copy(rows0, acc.at[dstb0], add=True)
                return carry

            lax.fori_loop(0, ch1, step, 0)

        plsc.subcore_barrier()
        pltpu.sync_copy(acc.at[pl.ds(s * ROWS, ROWS)],
                        psum.at[c].at[pl.ds(s * ROWS, ROWS)])

    return pl.kernel(
        body,
        out_type=jax.ShapeDtypeStruct((NC, N_PAD, D), jnp.float32),
        mesh=mesh,
        scratch_types=[
            pltpu.VMEM_SHARED((N_PAD, D), jnp.float32),
            pltpu.VMEM((chm, B), jnp.int32),
            pltpu.VMEM((B, D), jnp.float32),
            pltpu.VMEM((B, D), jnp.float32),
            pltpu.VMEM((B,), jnp.int32),
            pltpu.VMEM((B,), jnp.int32),
            pltpu.VMEM((B,), jnp.int32),
            pltpu.VMEM((B,), jnp.int32),
            pltpu.SemaphoreType.DMA,
            pltpu.SemaphoreType.DMA,
            pltpu.SemaphoreType.DMA,
            pltpu.SemaphoreType.DMA,
        ],
    )


CW = 64


def _make_counts(chunks):
    mesh = plsc.VectorSubcoreMesh(core_axis_name="c", subcore_axis_name="s")

    def body(dsti, zc, ones_in, pcnt, cacc, dstv, onesv):
        c = lax.axis_index("c")
        s = lax.axis_index("s")
        wid = c * NS + s
        pltpu.sync_copy(zc, cacc.at[pl.ds(s * ROWS, ROWS)])
        pltpu.sync_copy(ones_in, onesv)
        pltpu.sync_copy(dsti.at[wid], dstv)
        plsc.subcore_barrier()

        def step(j, carry):
            pltpu.sync_copy(onesv, cacc.at[dstv.at[j]], add=True)
            return carry

        lax.fori_loop(0, chunks, step, 0)
        plsc.subcore_barrier()
        pltpu.sync_copy(cacc.at[pl.ds(s * ROWS, ROWS)],
                        pcnt.at[c].at[pl.ds(s * ROWS, ROWS)])

    return pl.kernel(
        body,
        out_type=jax.ShapeDtypeStruct((NC, N_PAD, CW), jnp.float32),
        mesh=mesh,
        scratch_types=[
            pltpu.VMEM_SHARED((N_PAD, CW), jnp.float32),
            pltpu.VMEM((chunks, B), jnp.int32),
            pltpu.VMEM((B, CW), jnp.float32),
        ],
    )


def _dense_body(act, p0r, p1r, c0r, c1r, xr, wlr, blr, wrr, outr):
    cnt = c0r[0][:, 0:1] + c1r[0][:, 0:1]
    inv = 1.0 / jnp.maximum(cnt, 1.0)
    mean = (p0r[0] + p1r[0]) * inv
    y = (jnp.dot(mean, wlr[...], preferred_element_type=jnp.float32)
         + jnp.dot(xr[...], wrr[...], preferred_element_type=jnp.float32)
         + blr[...])
    if act:
        y = jnp.where(y > 0.0, y, jnp.exp(jnp.minimum(y, 0.0)) - 1.0)
    outr[...] = y


def _dense(psum, pcnt, x, Wl, bl, Wr, act):
    bn = 1000
    grid = (N // bn,)
    part0_spec = pl.BlockSpec((1, bn, D), lambda i: (0, i, 0))
    part1_spec = pl.BlockSpec((1, bn, D), lambda i: (1, i, 0))
    cnt0_spec = pl.BlockSpec((1, bn, CW), lambda i: (0, i, 0))
    cnt1_spec = pl.BlockSpec((1, bn, CW), lambda i: (1, i, 0))
    row_spec = pl.BlockSpec((bn, D), lambda i: (i, 0))
    w_spec = pl.BlockSpec((D, D), lambda i: (0, 0))
    b_spec = pl.BlockSpec((1, D), lambda i: (0, 0))
    return pl.pallas_call(
        functools.partial(_dense_body, act),
        grid=grid,
        in_specs=[part0_spec, part1_spec, cnt0_spec, cnt1_spec, row_spec,
                  w_spec, b_spec, w_spec],
        out_specs=row_spec,
        out_shape=jax.ShapeDtypeStruct((N, D), jnp.float32),
    )(psum, psum, pcnt, pcnt, x, Wl, bl.reshape(1, D), Wr)


def _pad_edges(src, dst, total):
    pad = total - src.shape[0]
    if pad:
        src = jnp.concatenate([src, jnp.zeros((pad,), jnp.int32)])
        dst = jnp.concatenate([dst, jnp.full((pad,), N, jnp.int32)])
    return src, dst


def _split_layout(arr, ch0, ch1):
    chm = max(ch0, ch1)
    n0 = NS * ch0 * B
    a0 = arr[:n0].reshape(NS, ch0, B)
    a1 = arr[n0:].reshape(NS, ch1, B)
    a0 = jnp.pad(a0, ((0, 0), (0, chm - ch0), (0, 0)))
    a1 = jnp.pad(a1, ((0, 0), (0, chm - ch1), (0, 0)))
    return jnp.concatenate([a0, a1], axis=0)


def kernel(x, edge_index, W1l, b1l, W1r, W2l, b2l, W2r):
    src = edge_index[0]
    dst = edge_index[1]
    e = src.shape[0]

    chunks_u = -(-e // (NW * B))
    src_u, dst_u = _pad_edges(src, dst, chunks_u * NW * B)
    dst3u = dst_u.reshape(NW, chunks_u, B)

    total_chunks = -(-e // (NS * B))
    ch0 = int(round(total_chunks * SPLIT))
    ch0 = min(max(2 * (ch0 // 2), 4), total_chunks - 1)
    ch1 = total_chunks - ch0
    src_a, dst_a = _pad_edges(src, dst, total_chunks * NS * B)
    packed = _split_layout(dst_a * 65536 + src_a, ch0, ch1)

    zf = jnp.zeros((ROWS, D), jnp.float32)
    zc = jnp.zeros((ROWS, CW), jnp.float32)
    ones = jnp.ones((B, CW), jnp.float32)

    aggregate = _make_aggregate(ch0, ch1)
    counts = _make_counts(chunks_u)

    pcnt = counts(dst3u, zc, ones)
    psum = aggregate(x, packed, zf)
    h = _dense(psum, pcnt, x, W1l, b1l, W1r, act=True)
    psum2 = aggregate(h, packed, zf)
    return _dense(psum2, pcnt, h, W2l, b2l, W2r, act=False)
